# baseline (speedup 1.0000x reference)
import jax
import jax.numpy as jnp
from jax import lax
from jax.experimental import pallas as pl
from jax.experimental.pallas import tpu as pltpu
from jax.experimental.pallas import tpu_sc as plsc

N_NODES = 10000
N_EDGES = 320000
D_FEAT = 128
DP = D_FEAT // 2

NC = 2
NS = 16
NW = NC * NS
EPW = N_EDGES // NW
CHUNK = 200
NCHUNK = EPW // CHUNK
GROUPS = -(-CHUNK // 16)
RPAD = GROUPS * 16 - CHUNK


def _sc_scores(h_hbm, src_hbm, dst_hbm, scores_hbm,
               idx_s, idx_d, srows0, drows0, srows1, drows1, scores_v,
               sem_i, sem_s0, sem_d0, sem_s1, sem_d1):
    wid = lax.axis_index("s") * NC + lax.axis_index("c")
    base = wid * EPW
    lanes = lax.iota(jnp.int32, 16)

    sbufs = (srows0, srows1)
    dbufs = (drows0, drows1)
    ssems = (sem_s0, sem_s1)
    dsems = (sem_d0, sem_d1)

    cp_is = pltpu.async_copy(src_hbm.at[pl.ds(base, EPW)], idx_s, sem_i)
    cp_id = pltpu.async_copy(dst_hbm.at[pl.ds(base, EPW)], idx_d, sem_i)
    cp_is.wait()
    cp_id.wait()

    def issue(cc, b):
        isl = idx_s.at[pl.ds(cc * CHUNK, CHUNK)]
        idl = idx_d.at[pl.ds(cc * CHUNK, CHUNK)]
        pltpu.async_copy(h_hbm.at[isl], sbufs[b].at[pl.ds(0, CHUNK)], ssems[b])
        pltpu.async_copy(h_hbm.at[idl], dbufs[b].at[pl.ds(0, CHUNK)], dsems[b])

    def drain(b):
        isl = idx_s.at[pl.ds(0, CHUNK)]
        idl = idx_d.at[pl.ds(0, CHUNK)]
        pltpu.make_async_copy(h_hbm.at[isl], sbufs[b].at[pl.ds(0, CHUNK)], ssems[b]).wait()
        pltpu.make_async_copy(h_hbm.at[idl], dbufs[b].at[pl.ds(0, CHUNK)], dsems[b]).wait()

    def compute(cc, b):
        sw = sbufs[b]
        dw = dbufs[b]

        def g_body(g, carry2):
            rows = lanes + g * 16
            accs = [jnp.zeros((16,), jnp.float32) for _ in range(4)]
            for d in range(DP):
                cols = (lanes + d) & (DP - 1)
                sv = plsc.load_gather(sw, [rows, cols])
                dv = plsc.load_gather(dw, [rows, cols])
                p = plsc.bitcast(sv, jnp.bfloat16) * plsc.bitcast(dv, jnp.bfloat16)
                a, bb = plsc.unpack(p, format=plsc.PackFormat.INTERLEAVED)
                accs[2 * (d % 2)] = accs[2 * (d % 2)] + a
                accs[2 * (d % 2) + 1] = accs[2 * (d % 2) + 1] + bb
            acc = (accs[0] + accs[1]) + (accs[2] + accs[3])
            scores_v[pl.ds(cc * CHUNK + g * 16, 16)] = acc
            return carry2

        lax.fori_loop(0, GROUPS, g_body, 0)

    issue(0, 0)

    def pair_body(i, carry):
        for b in range(2):
            cc = 2 * i + b
            nxt = jnp.minimum(cc + 1, NCHUNK - 1)
            issue(nxt, 1 - b)
            drain(b)
            compute(cc, b)
        return carry

    lax.fori_loop(0, NCHUNK // 2, pair_body, 0)
    drain(0)

    pltpu.sync_copy(scores_v.at[pl.ds(0, EPW)], scores_hbm.at[pl.ds(base, EPW)])


_sc_call = pl.kernel(
    _sc_scores,
    out_type=jax.ShapeDtypeStruct((N_EDGES,), jnp.float32),
    mesh=plsc.VectorSubcoreMesh(core_axis_name="c", subcore_axis_name="s"),
    compiler_params=pltpu.CompilerParams(needs_layout_passes=False,
                                         use_tc_tiling_on_sc=False),
    scratch_types=[
        pltpu.VMEM((EPW,), jnp.int32),
        pltpu.VMEM((EPW,), jnp.int32),
        pltpu.VMEM((CHUNK + RPAD, DP), jnp.int32),
        pltpu.VMEM((CHUNK + RPAD, DP), jnp.int32),
        pltpu.VMEM((CHUNK + RPAD, DP), jnp.int32),
        pltpu.VMEM((CHUNK + RPAD, DP), jnp.int32),
        pltpu.VMEM((EPW + RPAD * 16,), jnp.float32),
        pltpu.SemaphoreType.DMA,
        pltpu.SemaphoreType.DMA,
        pltpu.SemaphoreType.DMA,
        pltpu.SemaphoreType.DMA,
        pltpu.SemaphoreType.DMA,
    ],
)


def _norm_body(s_ref, o_ref):
    s = s_ref[...]
    mn = jnp.min(s)
    mx = jnp.max(s)
    o_ref[...] = jnp.where((s - mn) / (mx - mn) == 0.0, 0.0, 1.0)


def kernel(h, edge_index):
    ei = edge_index.astype(jnp.int32)
    hp = lax.bitcast_convert_type(
        h.astype(jnp.bfloat16).reshape(N_NODES, DP, 2), jnp.int32)
    scores = _sc_call(hp, ei[0], ei[1])
    s2d = scores.reshape(N_EDGES // 128, 128)
    out = pl.pallas_call(
        _norm_body,
        out_shape=jax.ShapeDtypeStruct(s2d.shape, jnp.float32),
    )(s2d)
    return out.reshape(N_EDGES, 1)


# speedup vs baseline: 1.3918x; 1.3918x over previous
import jax
import jax.numpy as jnp
from jax import lax
from jax.experimental import pallas as pl
from jax.experimental.pallas import tpu as pltpu
from jax.experimental.pallas import tpu_sc as plsc

N_NODES = 10000
N_EDGES = 320000
D_FEAT = 128
DP = D_FEAT // 2

NC = 2
NS = 16
NW = NC * NS
EPW = N_EDGES // NW
CHUNK = 200
NCHUNK = EPW // CHUNK
EUNROLL = 4


def _sc_scores(h_hbm, src_hbm, dst_hbm, scores_hbm,
               idx_s, idx_d, srows0, drows0, srows1, drows1, scores_v,
               sem_i, sem_s0, sem_d0, sem_s1, sem_d1):
    wid = lax.axis_index("s") * NC + lax.axis_index("c")
    base = wid * EPW
    lanes = lax.iota(jnp.int32, 16)

    sbufs = (srows0, srows1)
    dbufs = (drows0, drows1)
    ssems = (sem_s0, sem_s1)
    dsems = (sem_d0, sem_d1)

    cp_is = pltpu.async_copy(src_hbm.at[pl.ds(base, EPW)], idx_s, sem_i)
    cp_id = pltpu.async_copy(dst_hbm.at[pl.ds(base, EPW)], idx_d, sem_i)
    cp_is.wait()
    cp_id.wait()

    def issue(cc, b):
        isl = idx_s.at[pl.ds(cc * CHUNK, CHUNK)]
        idl = idx_d.at[pl.ds(cc * CHUNK, CHUNK)]
        pltpu.async_copy(h_hbm.at[isl], sbufs[b], ssems[b])
        pltpu.async_copy(h_hbm.at[idl], dbufs[b], dsems[b])

    def drain(b):
        isl = idx_s.at[pl.ds(0, CHUNK)]
        idl = idx_d.at[pl.ds(0, CHUNK)]
        pltpu.make_async_copy(h_hbm.at[isl], sbufs[b], ssems[b]).wait()
        pltpu.make_async_copy(h_hbm.at[idl], dbufs[b], dsems[b]).wait()

    def compute(cc, b):
        sw = sbufs[b]
        dw = dbufs[b]

        def e_body(i, carry2):
            for j in range(EUNROLL):
                e = i * EUNROLL + j
                accs = [jnp.zeros((16,), jnp.float32) for _ in range(2)]
                for k in range(DP // 16):
                    svk = sw[e, pl.ds(k * 16, 16)]
                    dvk = dw[e, pl.ds(k * 16, 16)]
                    p = (plsc.bitcast(svk, jnp.bfloat16)
                         * plsc.bitcast(dvk, jnp.bfloat16))
                    a, bb = plsc.unpack(p, format=plsc.PackFormat.INTERLEAVED)
                    accs[0] = accs[0] + a
                    accs[1] = accs[1] + bb
                s = plsc.cumsum(accs[0] + accs[1])
                plsc.store_scatter(
                    scores_v, [jnp.full((16,), cc * CHUNK + e, jnp.int32)],
                    s, mask=lanes == 15)
            return carry2

        lax.fori_loop(0, CHUNK // EUNROLL, e_body, 0)

    issue(0, 0)

    def pair_body(i, carry):
        for b in range(2):
            cc = 2 * i + b
            nxt = jnp.minimum(cc + 1, NCHUNK - 1)
            issue(nxt, 1 - b)
            drain(b)
            compute(cc, b)
        return carry

    lax.fori_loop(0, NCHUNK // 2, pair_body, 0)
    drain(0)

    pltpu.sync_copy(scores_v, scores_hbm.at[pl.ds(base, EPW)])


_sc_call = pl.kernel(
    _sc_scores,
    out_type=jax.ShapeDtypeStruct((N_EDGES,), jnp.float32),
    mesh=plsc.VectorSubcoreMesh(core_axis_name="c", subcore_axis_name="s"),
    compiler_params=pltpu.CompilerParams(needs_layout_passes=False,
                                         use_tc_tiling_on_sc=False),
    scratch_types=[
        pltpu.VMEM((EPW,), jnp.int32),
        pltpu.VMEM((EPW,), jnp.int32),
        pltpu.VMEM((CHUNK, DP), jnp.int32),
        pltpu.VMEM((CHUNK, DP), jnp.int32),
        pltpu.VMEM((CHUNK, DP), jnp.int32),
        pltpu.VMEM((CHUNK, DP), jnp.int32),
        pltpu.VMEM((EPW,), jnp.float32),
        pltpu.SemaphoreType.DMA,
        pltpu.SemaphoreType.DMA,
        pltpu.SemaphoreType.DMA,
        pltpu.SemaphoreType.DMA,
        pltpu.SemaphoreType.DMA,
    ],
)


def _norm_body(s_ref, o_ref):
    s = s_ref[...]
    mn = jnp.min(s)
    mx = jnp.max(s)
    o_ref[...] = jnp.where((s - mn) / (mx - mn) == 0.0, 0.0, 1.0)


def kernel(h, edge_index):
    ei = edge_index.astype(jnp.int32)
    hp = lax.bitcast_convert_type(
        h.astype(jnp.bfloat16).reshape(N_NODES, DP, 2), jnp.int32)
    scores = _sc_call(hp, ei[0], ei[1])
    s2d = scores.reshape(N_EDGES // 128, 128)
    out = pl.pallas_call(
        _norm_body,
        out_shape=jax.ShapeDtypeStruct(s2d.shape, jnp.float32),
    )(s2d)
    return out.reshape(N_EDGES, 1)
